# baseline (speedup 1.0000x reference)
import numpy as np
import jax
import jax.numpy as jnp
from jax.experimental import pallas as pl
from jax.experimental.pallas import tpu as pltpu

_A1, _A2, _A3 = -4.0 / 15.0, -2.0 / 15.0, -1.0 / 15.0
_AM = -1.0 / 30.0
_EPS = 4e-5


def _band_weights(w: int) -> np.ndarray:
    eye0 = np.eye(w, dtype=np.float32)
    eyep = np.eye(w, k=1, dtype=np.float32)
    eyem = np.eye(w, k=-1, dtype=np.float32)
    t_top = _A1 * eyep + _A2 * eye0 + _A3 * eyem
    t_mid = _AM * eyep + 1.0 * eye0 + _AM * eyem
    t_bot = _A3 * eyep + _A2 * eye0 + _A1 * eyem
    return np.concatenate([t_top, t_mid, t_bot], axis=0)


def _plane_kernel(x_ref, t_ref, o_ref):
    x = x_ref[...]
    N, H, W = x.shape
    zrow = jnp.zeros((N, 1, W), x.dtype)
    xd = jnp.concatenate([zrow, x[:, :H - 1, :]], axis=1)
    xu = jnp.concatenate([x[:, 1:, :], zrow], axis=1)
    stack = jnp.concatenate(
        [xd.astype(jnp.bfloat16), x.astype(jnp.bfloat16),
         xu.astype(jnp.bfloat16)], axis=2).reshape(N * H, 3 * W)
    s = jnp.dot(stack, t_ref[...], preferred_element_type=jnp.float32)
    s = s.reshape(N, H, W)
    m = jnp.mean(s, axis=(1, 2), keepdims=True)
    v = jnp.mean(s * s, axis=(1, 2), keepdims=True) - m * m
    o_ref[...] = (s - m) * jax.lax.rsqrt(v + _EPS)


def kernel(x):
    B, C, H, W = x.shape
    P = B * C
    N = 4
    xf = x.reshape(P, H, W)
    t = jnp.asarray(_band_weights(W), dtype=jnp.bfloat16)
    out = pl.pallas_call(
        _plane_kernel,
        grid=(P // N,),
        in_specs=[
            pl.BlockSpec((N, H, W), lambda i: (i, 0, 0)),
            pl.BlockSpec((3 * W, W), lambda i: (0, 0)),
        ],
        out_specs=pl.BlockSpec((N, H, W), lambda i: (i, 0, 0)),
        out_shape=jax.ShapeDtypeStruct((P, H, W), x.dtype),
        compiler_params=pltpu.CompilerParams(
            dimension_semantics=("parallel",),
        ),
    )(xf, t)
    return out.reshape(B, C, H, W)


# speedup vs baseline: 1.0570x; 1.0570x over previous
import numpy as np
import jax
import jax.numpy as jnp
from jax.experimental import pallas as pl
from jax.experimental.pallas import tpu as pltpu

_A1, _A2, _A3 = -4.0 / 15.0, -2.0 / 15.0, -1.0 / 15.0
_AM = -1.0 / 30.0
_EPS = 4e-5


def _band_weights(w: int) -> np.ndarray:
    eye0 = np.eye(w, dtype=np.float32)
    eyep = np.eye(w, k=1, dtype=np.float32)
    eyem = np.eye(w, k=-1, dtype=np.float32)
    t_top = _A1 * eyep + _A2 * eye0 + _A3 * eyem
    t_mid = _AM * eyep + 1.0 * eye0 + _AM * eyem
    t_bot = _A3 * eyep + _A2 * eye0 + _A1 * eyem
    return np.concatenate([t_top, t_mid, t_bot], axis=0)


def _plane_kernel(x_ref, t_ref, o_ref):
    N, H, W = x_ref.shape
    t = t_ref[...]
    bf = jnp.bfloat16
    f = jnp.float32
    for p in range(N):
        x = x_ref[p]
        zrow = jnp.zeros((1, W), x.dtype)
        xd = jnp.concatenate([zrow, x[:H - 1]], axis=0)
        xu = jnp.concatenate([x[1:], zrow], axis=0)
        stack = jnp.concatenate(
            [xd.astype(bf), x.astype(bf), xu.astype(bf)], axis=1)
        s = jnp.dot(stack, t, preferred_element_type=f)
        m = jnp.mean(s)
        v = jnp.mean(s * s) - m * m
        o_ref[p] = (s - m) * jax.lax.rsqrt(v + _EPS)


def kernel(x):
    B, C, H, W = x.shape
    P = B * C
    N = 8
    xf = x.reshape(P, H, W)
    t = jnp.asarray(_band_weights(W), dtype=jnp.bfloat16)
    out = pl.pallas_call(
        _plane_kernel,
        grid=(P // N,),
        in_specs=[
            pl.BlockSpec((N, H, W), lambda i: (i, 0, 0)),
            pl.BlockSpec((3 * W, W), lambda i: (0, 0)),
        ],
        out_specs=pl.BlockSpec((N, H, W), lambda i: (i, 0, 0)),
        out_shape=jax.ShapeDtypeStruct((P, H, W), x.dtype),
        compiler_params=pltpu.CompilerParams(
            dimension_semantics=("parallel",),
        ),
    )(xf, t)
    return out.reshape(B, C, H, W)
